# baseline (speedup 1.0000x reference)
import functools

import jax
import jax.numpy as jnp
from jax import lax
from jax.experimental import pallas as pl
from jax.experimental.pallas import tpu as pltpu
from jax.experimental.pallas import tpu_sc as plsc

_NC = 2
_NS = 16
_NW = _NC * _NS
_D = 64
_CHUNK = 128


def _sc_gather(B):
    b_per_w = B // _NW
    n_chunks = b_per_w // _CHUNK
    mesh = plsc.VectorSubcoreMesh(core_axis_name="c", subcore_axis_name="s")
    rows_t = jax.ShapeDtypeStruct((B, _D), jnp.float32)

    @functools.partial(
        pl.kernel,
        mesh=mesh,
        compiler_params=pltpu.CompilerParams(use_tc_tiling_on_sc=False),
        out_type=(rows_t, rows_t, rows_t),
        scratch_types=[
            pltpu.VMEM((n_chunks, _CHUNK), jnp.int32),
            pltpu.VMEM((n_chunks, _CHUNK), jnp.int32),
            pltpu.VMEM((n_chunks, _CHUNK), jnp.int32),
            pltpu.VMEM((b_per_w, _D), jnp.float32),
            pltpu.VMEM((b_per_w, _D), jnp.float32),
            pltpu.VMEM((b_per_w, _D), jnp.float32),
            pltpu.SemaphoreType.DMA,
        ],
    )
    def gather_kernel(ent_hbm, rel_hbm, hidx_hbm, tidx_hbm, ridx_hbm,
                      ho_hbm, to_hbm, ro_hbm,
                      hi_v, ti_v, ri_v, hr_v, tr_v, rr_v, sem):
        wid = lax.axis_index("s") * _NC + lax.axis_index("c")
        base = wid * b_per_w
        for c in range(n_chunks):
            src = pl.ds(base + c * _CHUNK, _CHUNK)
            pltpu.sync_copy(hidx_hbm.at[src], hi_v.at[c])
            pltpu.sync_copy(tidx_hbm.at[src], ti_v.at[c])
            pltpu.sync_copy(ridx_hbm.at[src], ri_v.at[c])
        copies = []
        for c in range(n_chunks):
            dst = pl.ds(c * _CHUNK, _CHUNK)
            copies.append(pltpu.async_copy(ent_hbm.at[hi_v.at[c]], hr_v.at[dst], sem))
            copies.append(pltpu.async_copy(ent_hbm.at[ti_v.at[c]], tr_v.at[dst], sem))
            copies.append(pltpu.async_copy(rel_hbm.at[ri_v.at[c]], rr_v.at[dst], sem))
        for cp in copies:
            cp.wait()
        out = pl.ds(base, b_per_w)
        pltpu.sync_copy(hr_v, ho_hbm.at[out])
        pltpu.sync_copy(tr_v, to_hbm.at[out])
        pltpu.sync_copy(rr_v, ro_hbm.at[out])

    return gather_kernel


def _tc_score_body(h_ref, t_ref, r_ref, o_ref):
    o_ref[...] = jnp.sum(jnp.abs(h_ref[...] + r_ref[...] - t_ref[...]), axis=1)


def _tc_score(h, t, r):
    B = h.shape[0]
    return pl.pallas_call(
        _tc_score_body,
        out_shape=jax.ShapeDtypeStruct((B,), jnp.float32),
    )(h, t, r)


def kernel(entity_emb, relation_emb, pos_h, pos_r, pos_t):
    B = pos_h.shape[0]
    h_rows, t_rows, r_rows = _sc_gather(B)(
        entity_emb, relation_emb, pos_h, pos_t, pos_r)
    return _tc_score(h_rows, t_rows, r_rows)


# speedup vs baseline: 2.0235x; 2.0235x over previous
import dataclasses
import functools

import jax
import jax.numpy as jnp
from jax import lax
from jax.experimental import pallas as pl
from jax.experimental.pallas import tpu as pltpu
from jax.experimental.pallas import tpu_sc as plsc

_NC = 2
_NS = 16
_NW = _NC * _NS
_D = 64
_L = 16
_CHUNK = 128
_TBL = 8192


def _tc_transpose_body(in_ref, out_ref):
    h = _TBL // 2
    out_ref[:, 0:_D] = in_ref[:, 0:h][...].T
    out_ref[:, _D:2 * _D] = in_ref[:, h:_TBL][...].T


def _tc_pack_transpose(ent_t):
    d, n = ent_t.shape
    n_blocks = (n + _TBL - 1) // _TBL
    return pl.pallas_call(
        _tc_transpose_body,
        grid=(n_blocks,),
        in_specs=[pl.BlockSpec((d, _TBL), lambda i: (0, i))],
        out_specs=pl.BlockSpec((_TBL // 2, 2 * d), lambda i: (i, 0)),
        out_shape=jax.ShapeDtypeStruct((n_blocks * (_TBL // 2), 2 * d),
                                       jnp.float32),
    )(ent_t)


def _sc_score(B):
    b_per_w = B // _NW
    n_chunks = b_per_w // _CHUNK
    n_groups = _CHUNK // _L
    mesh = plsc.VectorSubcoreMesh(core_axis_name="c", subcore_axis_name="s")

    cp = pltpu.CompilerParams(use_tc_tiling_on_sc=False)
    if "needs_layout_passes" in pltpu.CompilerParams.__dataclass_fields__:
        cp = dataclasses.replace(cp, needs_layout_passes=False)

    @functools.partial(
        pl.kernel,
        mesh=mesh,
        compiler_params=cp,
        out_type=jax.ShapeDtypeStruct((B,), jnp.float32),
        scratch_types=[
            pltpu.VMEM((b_per_w,), jnp.int32),
            pltpu.VMEM((b_per_w,), jnp.int32),
            pltpu.VMEM((b_per_w,), jnp.int32),
            pltpu.VMEM((b_per_w,), jnp.int32),
            pltpu.VMEM((b_per_w,), jnp.int32),
            pltpu.VMEM((_CHUNK, 2 * _D), jnp.float32),
            pltpu.VMEM((_CHUNK, 2 * _D), jnp.float32),
            pltpu.VMEM((_CHUNK, _D), jnp.float32),
            pltpu.VMEM((b_per_w,), jnp.float32),
            pltpu.SemaphoreType.DMA,
        ],
    )
    def score_kernel(ent_hbm, rel_hbm, hidx_hbm, tidx_hbm, ridx_hbm, out_hbm,
                     hi_v, ti_v, ri_v, hp_v, tp_v,
                     hrow_v, trow_v, rrow_v, out_v, sem):
        wid = lax.axis_index("s") * _NC + lax.axis_index("c")
        base = wid * b_per_w
        src = pl.ds(base, b_per_w)
        pltpu.sync_copy(hidx_hbm.at[src], hi_v)
        pltpu.sync_copy(tidx_hbm.at[src], ti_v)
        pltpu.sync_copy(ridx_hbm.at[src], ri_v)
        for s in range(b_per_w // _L):
            sl = pl.ds(s * _L, _L)
            hi = hi_v[sl]
            ti = ti_v[sl]
            hp_v[sl] = lax.bitwise_or(
                lax.shift_left(lax.shift_right_logical(hi, 13), 12),
                lax.bitwise_and(hi, 4095))
            tp_v[sl] = lax.bitwise_or(
                lax.shift_left(lax.shift_right_logical(ti, 13), 12),
                lax.bitwise_and(ti, 4095))

        iota = lax.iota(jnp.int32, _L)

        @pl.loop(0, n_chunks)
        def _(c):
            csl = pl.ds(c * _CHUNK, _CHUNK)
            cph = pltpu.async_copy(ent_hbm.at[hp_v.at[csl]], hrow_v, sem)
            cpt = pltpu.async_copy(ent_hbm.at[tp_v.at[csl]], trow_v, sem)
            cpr = pltpu.async_copy(rel_hbm.at[ri_v.at[csl]], rrow_v, sem)
            cph.wait()
            cpt.wait()
            cpr.wait()
            for g in range(n_groups):
                lanes = iota + g * _L
                rsl = pl.ds(c * _CHUNK + g * _L, _L)
                hcol = lax.shift_left(
                    lax.bitwise_and(lax.shift_right_logical(hi_v[rsl], 12), 1), 6)
                tcol = lax.shift_left(
                    lax.bitwise_and(lax.shift_right_logical(ti_v[rsl], 12), 1), 6)
                acc = jnp.zeros((_L,), jnp.float32)
                for d in range(_D):
                    dv = jnp.full((_L,), d, jnp.int32)
                    hv = plsc.load_gather(hrow_v, [lanes, hcol + d])
                    tv = plsc.load_gather(trow_v, [lanes, tcol + d])
                    rv = plsc.load_gather(rrow_v, [lanes, dv])
                    acc = acc + jnp.abs(hv + rv - tv)
                out_v[rsl] = acc

        pltpu.sync_copy(out_v, out_hbm.at[pl.ds(base, b_per_w)])

    return score_kernel


def kernel(entity_emb, relation_emb, pos_h, pos_r, pos_t):
    B = pos_h.shape[0]
    ent_packed = _tc_pack_transpose(entity_emb.T)
    return _sc_score(B)(ent_packed, relation_emb, pos_h, pos_t, pos_r)


# speedup vs baseline: 2.2416x; 1.1078x over previous
import dataclasses
import functools

import jax
import jax.numpy as jnp
from jax import lax
from jax.experimental import pallas as pl
from jax.experimental.pallas import tpu as pltpu
from jax.experimental.pallas import tpu_sc as plsc

_NC = 2
_NS = 16
_NW = _NC * _NS
_D = 64
_L = 16
_CHUNK = 128
_TBL = 16384
_TSH = 13
_EPAD = 144
_RPAD = 80


def _tc_transpose_body(in_ref, out_ref):
    h = _TBL // 2
    a = in_ref[:, 0:h][...].T
    b = in_ref[:, h:_TBL][...].T
    out_ref[...] = jnp.concatenate([a, b], axis=1)


def _tc_transpose(ent_t):
    d, n = ent_t.shape
    n_blocks = (n + _TBL - 1) // _TBL
    return pl.pallas_call(
        _tc_transpose_body,
        grid=(n_blocks,),
        in_specs=[pl.BlockSpec((d, _TBL), lambda i: (0, i))],
        out_specs=pl.BlockSpec((_TBL // 2, 2 * d), lambda i: (i, 0)),
        out_shape=jax.ShapeDtypeStruct((n_blocks * (_TBL // 2), 2 * d),
                                       jnp.float32),
    )(ent_t)


def _sc_score(B):
    b_per_w = B // _NW
    n_chunks = b_per_w // _CHUNK
    n_groups = _CHUNK // _L
    mesh = plsc.VectorSubcoreMesh(core_axis_name="c", subcore_axis_name="s")

    cp = pltpu.CompilerParams(use_tc_tiling_on_sc=False)
    if "needs_layout_passes" in pltpu.CompilerParams.__dataclass_fields__:
        cp = dataclasses.replace(cp, needs_layout_passes=False)

    @functools.partial(
        pl.kernel,
        mesh=mesh,
        compiler_params=cp,
        out_type=jax.ShapeDtypeStruct((B,), jnp.float32),
        scratch_types=[
            pltpu.VMEM((b_per_w,), jnp.int32),
            pltpu.VMEM((b_per_w,), jnp.int32),
            pltpu.VMEM((b_per_w,), jnp.int32),
            pltpu.VMEM((b_per_w,), jnp.int32),
            pltpu.VMEM((b_per_w,), jnp.int32),
            pltpu.VMEM((_CHUNK, 2 * _D), jnp.float32),
            pltpu.VMEM((_CHUNK, 2 * _D), jnp.float32),
            pltpu.VMEM((_CHUNK, _D), jnp.float32),
            pltpu.VMEM((b_per_w,), jnp.float32),
            pltpu.SemaphoreType.DMA,
        ],
    )
    def score_kernel(ent_hbm, rel_hbm, hidx_hbm, tidx_hbm, ridx_hbm, out_hbm,
                     hi_v, ti_v, ri_v, hp_v, tp_v,
                     hrow_v, trow_v, rrow_v, out_v, sem):
        wid = lax.axis_index("s") * _NC + lax.axis_index("c")
        base = wid * b_per_w
        src = pl.ds(base, b_per_w)
        pltpu.sync_copy(hidx_hbm.at[src], hi_v)
        pltpu.sync_copy(tidx_hbm.at[src], ti_v)
        pltpu.sync_copy(ridx_hbm.at[src], ri_v)
        for s in range(b_per_w // _L):
            sl = pl.ds(s * _L, _L)
            hi = hi_v[sl]
            ti = ti_v[sl]
            hp_v[sl] = lax.bitwise_or(
                lax.shift_left(lax.shift_right_logical(hi, _TSH + 1), _TSH),
                lax.bitwise_and(hi, (1 << _TSH) - 1))
            tp_v[sl] = lax.bitwise_or(
                lax.shift_left(lax.shift_right_logical(ti, _TSH + 1), _TSH),
                lax.bitwise_and(ti, (1 << _TSH) - 1))

        iota = lax.iota(jnp.int32, _L)

        @pl.loop(0, n_chunks)
        def _(c):
            csl = pl.ds(c * _CHUNK, _CHUNK)
            cph = pltpu.async_copy(ent_hbm.at[hp_v.at[csl]], hrow_v, sem)
            cpt = pltpu.async_copy(ent_hbm.at[tp_v.at[csl]], trow_v, sem)
            cpr = pltpu.async_copy(rel_hbm.at[ri_v.at[csl]], rrow_v, sem)
            cph.wait()
            cpt.wait()
            cpr.wait()
            for g in range(n_groups):
                lanes = iota + g * _L
                rsl = pl.ds(c * _CHUNK + g * _L, _L)
                hcol = lax.shift_left(
                    lax.bitwise_and(lax.shift_right_logical(hi_v[rsl], _TSH), 1), 6)
                tcol = lax.shift_left(
                    lax.bitwise_and(lax.shift_right_logical(ti_v[rsl], _TSH), 1), 6)
                acc = jnp.zeros((_L,), jnp.float32)
                for d in range(_D):
                    dv = jnp.full((_L,), d, jnp.int32)
                    hv = plsc.load_gather(hrow_v, [lanes, hcol + d])
                    tv = plsc.load_gather(trow_v, [lanes, tcol + d])
                    rv = plsc.load_gather(rrow_v, [lanes, dv])
                    acc = acc + jnp.abs(hv + rv - tv)
                out_v[rsl] = acc

        pltpu.sync_copy(out_v, out_hbm.at[pl.ds(base, b_per_w)])

    return score_kernel


def kernel(entity_emb, relation_emb, pos_h, pos_r, pos_t):
    B = pos_h.shape[0]
    ent_wide = _tc_transpose(entity_emb.T)
    return _sc_score(B)(ent_wide, relation_emb, pos_h, pos_t, pos_r)


# speedup vs baseline: 2.3607x; 1.0531x over previous
import dataclasses
import functools

import jax
import jax.numpy as jnp
from jax import lax
from jax.experimental import pallas as pl
from jax.experimental.pallas import tpu as pltpu
from jax.experimental.pallas import tpu_sc as plsc

_NC = 2
_NS = 16
_NW = _NC * _NS
_D = 64
_L = 16
_CHUNK = 128
_TBL = 32768
_TSH = 14
_EPAD = 144
_RPAD = 80


def _tc_transpose_body(in_ref, out_ref):
    h = _TBL // 2
    a = in_ref[:, 0:h][...].T
    b = in_ref[:, h:_TBL][...].T
    out_ref[...] = jnp.concatenate([a, b], axis=1)


def _tc_transpose(ent_t):
    d, n = ent_t.shape
    n_blocks = (n + _TBL - 1) // _TBL
    return pl.pallas_call(
        _tc_transpose_body,
        grid=(n_blocks,),
        in_specs=[pl.BlockSpec((d, _TBL), lambda i: (0, i))],
        out_specs=pl.BlockSpec((_TBL // 2, 2 * d), lambda i: (i, 0)),
        out_shape=jax.ShapeDtypeStruct((n_blocks * (_TBL // 2), 2 * d),
                                       jnp.float32),
    )(ent_t)


def _sc_score(B):
    b_per_w = B // _NW
    n_chunks = b_per_w // _CHUNK
    n_groups = _CHUNK // _L
    mesh = plsc.VectorSubcoreMesh(core_axis_name="c", subcore_axis_name="s")

    cp = pltpu.CompilerParams(use_tc_tiling_on_sc=False)
    if "needs_layout_passes" in pltpu.CompilerParams.__dataclass_fields__:
        cp = dataclasses.replace(cp, needs_layout_passes=False)

    @functools.partial(
        pl.kernel,
        mesh=mesh,
        compiler_params=cp,
        out_type=jax.ShapeDtypeStruct((B,), jnp.float32),
        scratch_types=[
            pltpu.VMEM((b_per_w,), jnp.int32),
            pltpu.VMEM((b_per_w,), jnp.int32),
            pltpu.VMEM((b_per_w,), jnp.int32),
            pltpu.VMEM((b_per_w,), jnp.int32),
            pltpu.VMEM((b_per_w,), jnp.int32),
            pltpu.VMEM((_CHUNK, 2 * _D), jnp.float32),
            pltpu.VMEM((_CHUNK, 2 * _D), jnp.float32),
            pltpu.VMEM((_CHUNK, _D), jnp.float32),
            pltpu.VMEM((b_per_w,), jnp.float32),
            pltpu.SemaphoreType.DMA,
        ],
    )
    def score_kernel(ent_hbm, rel_hbm, hidx_hbm, tidx_hbm, ridx_hbm, out_hbm,
                     hi_v, ti_v, ri_v, hp_v, tp_v,
                     hrow_v, trow_v, rrow_v, out_v, sem):
        wid = lax.axis_index("s") * _NC + lax.axis_index("c")
        base = wid * b_per_w
        src = pl.ds(base, b_per_w)
        pltpu.sync_copy(hidx_hbm.at[src], hi_v)
        pltpu.sync_copy(tidx_hbm.at[src], ti_v)
        pltpu.sync_copy(ridx_hbm.at[src], ri_v)
        for s in range(b_per_w // _L):
            sl = pl.ds(s * _L, _L)
            hi = hi_v[sl]
            ti = ti_v[sl]
            hp_v[sl] = lax.bitwise_or(
                lax.shift_left(lax.shift_right_logical(hi, _TSH + 1), _TSH),
                lax.bitwise_and(hi, (1 << _TSH) - 1))
            tp_v[sl] = lax.bitwise_or(
                lax.shift_left(lax.shift_right_logical(ti, _TSH + 1), _TSH),
                lax.bitwise_and(ti, (1 << _TSH) - 1))

        iota = lax.iota(jnp.int32, _L)

        @pl.loop(0, n_chunks)
        def _(c):
            csl = pl.ds(c * _CHUNK, _CHUNK)
            cph = pltpu.async_copy(ent_hbm.at[hp_v.at[csl]], hrow_v, sem)
            cpt = pltpu.async_copy(ent_hbm.at[tp_v.at[csl]], trow_v, sem)
            cpr = pltpu.async_copy(rel_hbm.at[ri_v.at[csl]], rrow_v, sem)
            cph.wait()
            cpt.wait()
            cpr.wait()
            for g in range(n_groups):
                lanes = iota + g * _L
                rsl = pl.ds(c * _CHUNK + g * _L, _L)
                hcol = lax.shift_left(
                    lax.bitwise_and(lax.shift_right_logical(hi_v[rsl], _TSH), 1), 6)
                tcol = lax.shift_left(
                    lax.bitwise_and(lax.shift_right_logical(ti_v[rsl], _TSH), 1), 6)
                acc = jnp.zeros((_L,), jnp.float32)
                for d in range(_D):
                    dv = jnp.full((_L,), d, jnp.int32)
                    hv = plsc.load_gather(hrow_v, [lanes, hcol + d])
                    tv = plsc.load_gather(trow_v, [lanes, tcol + d])
                    rv = plsc.load_gather(rrow_v, [lanes, dv])
                    acc = acc + jnp.abs(hv + rv - tv)
                out_v[rsl] = acc

        pltpu.sync_copy(out_v, out_hbm.at[pl.ds(base, b_per_w)])

    return score_kernel


def kernel(entity_emb, relation_emb, pos_h, pos_r, pos_t):
    B = pos_h.shape[0]
    ent_wide = _tc_transpose(entity_emb.T)
    return _sc_score(B)(ent_wide, relation_emb, pos_h, pos_t, pos_r)


# speedup vs baseline: 2.3705x; 1.0041x over previous
import dataclasses
import functools

import jax
import jax.numpy as jnp
from jax import lax
from jax.experimental import pallas as pl
from jax.experimental.pallas import tpu as pltpu
from jax.experimental.pallas import tpu_sc as plsc

_NC = 2
_NS = 16
_NW = _NC * _NS
_D = 64
_L = 16
_CHUNK = 128
_TBL = 32768
_TSH = 14
_EPAD = 144
_RPAD = 80


def _tc_transpose_body(in_ref, out_ref):
    h = _TBL // 2
    a = in_ref[:, 0:h][...].T
    b = in_ref[:, h:_TBL][...].T
    out_ref[...] = jnp.concatenate([a, b], axis=1)


def _tc_transpose(ent_t):
    d, n = ent_t.shape
    n_blocks = (n + _TBL - 1) // _TBL
    return pl.pallas_call(
        _tc_transpose_body,
        grid=(n_blocks,),
        in_specs=[pl.BlockSpec((d, _TBL), lambda i: (0, i))],
        out_specs=pl.BlockSpec((_TBL // 2, 2 * d), lambda i: (i, 0)),
        out_shape=jax.ShapeDtypeStruct((n_blocks * (_TBL // 2), 2 * d),
                                       jnp.float32),
    )(ent_t)


def _sc_score(B):
    b_per_w = B // _NW
    n_chunks = b_per_w // _CHUNK
    n_groups = _CHUNK // _L
    mesh = plsc.VectorSubcoreMesh(core_axis_name="c", subcore_axis_name="s")

    cp = pltpu.CompilerParams(use_tc_tiling_on_sc=False)
    if "needs_layout_passes" in pltpu.CompilerParams.__dataclass_fields__:
        cp = dataclasses.replace(cp, needs_layout_passes=False)

    @functools.partial(
        pl.kernel,
        mesh=mesh,
        compiler_params=cp,
        out_type=jax.ShapeDtypeStruct((B,), jnp.float32),
        scratch_types=[
            pltpu.VMEM((b_per_w,), jnp.int32),
            pltpu.VMEM((b_per_w,), jnp.int32),
            pltpu.VMEM((b_per_w,), jnp.int32),
            pltpu.VMEM((b_per_w,), jnp.int32),
            pltpu.VMEM((b_per_w,), jnp.int32),
            pltpu.VMEM((_CHUNK, 2 * _D), jnp.float32),
            pltpu.VMEM((_CHUNK, 2 * _D), jnp.float32),
            pltpu.VMEM((_CHUNK, _D), jnp.float32),
            pltpu.VMEM((_CHUNK, 2 * _D), jnp.float32),
            pltpu.VMEM((_CHUNK, 2 * _D), jnp.float32),
            pltpu.VMEM((_CHUNK, _D), jnp.float32),
            pltpu.VMEM((b_per_w,), jnp.float32),
            pltpu.SemaphoreType.DMA,
            pltpu.SemaphoreType.DMA,
        ],
    )
    def score_kernel(ent_hbm, rel_hbm, hidx_hbm, tidx_hbm, ridx_hbm, out_hbm,
                     hi_v, ti_v, ri_v, hp_v, tp_v,
                     hrow0_v, trow0_v, rrow0_v, hrow1_v, trow1_v, rrow1_v,
                     out_v, sem0, sem1):
        wid = lax.axis_index("s") * _NC + lax.axis_index("c")
        base = wid * b_per_w
        src = pl.ds(base, b_per_w)
        pltpu.sync_copy(hidx_hbm.at[src], hi_v)
        pltpu.sync_copy(tidx_hbm.at[src], ti_v)
        pltpu.sync_copy(ridx_hbm.at[src], ri_v)
        for s in range(b_per_w // _L):
            sl = pl.ds(s * _L, _L)
            hi = hi_v[sl]
            ti = ti_v[sl]
            hp_v[sl] = lax.bitwise_or(
                lax.shift_left(lax.shift_right_logical(hi, _TSH + 1), _TSH),
                lax.bitwise_and(hi, (1 << _TSH) - 1))
            tp_v[sl] = lax.bitwise_or(
                lax.shift_left(lax.shift_right_logical(ti, _TSH + 1), _TSH),
                lax.bitwise_and(ti, (1 << _TSH) - 1))

        iota = lax.iota(jnp.int32, _L)
        bufs = ((hrow0_v, trow0_v, rrow0_v), (hrow1_v, trow1_v, rrow1_v))

        def fire(c, buf, sem):
            csl = pl.ds(c * _CHUNK, _CHUNK)
            pltpu.async_copy(ent_hbm.at[hp_v.at[csl]], buf[0], sem)
            pltpu.async_copy(ent_hbm.at[tp_v.at[csl]], buf[1], sem)
            pltpu.async_copy(rel_hbm.at[ri_v.at[csl]], buf[2], sem)

        def compute(c, buf):
            hrow_v, trow_v, rrow_v = buf
            for g in range(n_groups):
                lanes = iota + g * _L
                rsl = pl.ds(c * _CHUNK + g * _L, _L)
                hcol = lax.shift_left(
                    lax.bitwise_and(lax.shift_right_logical(hi_v[rsl], _TSH), 1), 6)
                tcol = lax.shift_left(
                    lax.bitwise_and(lax.shift_right_logical(ti_v[rsl], _TSH), 1), 6)
                acc = jnp.zeros((_L,), jnp.float32)
                for d in range(_D):
                    dv = jnp.full((_L,), d, jnp.int32)
                    hv = plsc.load_gather(hrow_v, [lanes, hcol + d])
                    tv = plsc.load_gather(trow_v, [lanes, tcol + d])
                    rv = plsc.load_gather(rrow_v, [lanes, dv])
                    acc = acc + jnp.abs(hv + rv - tv)
                out_v[rsl] = acc

        def drain(buf, sem):
            pltpu.make_async_copy(ent_hbm.at[pl.ds(0, _CHUNK)], buf[0], sem).wait()
            pltpu.make_async_copy(ent_hbm.at[pl.ds(0, _CHUNK)], buf[1], sem).wait()
            pltpu.make_async_copy(rel_hbm.at[pl.ds(0, _CHUNK)], buf[2], sem).wait()

        fire(0, bufs[0], sem0)
        fire(1, bufs[1], sem1)

        @pl.loop(0, n_chunks, step=2)
        def _(c):
            drain(bufs[0], sem0)
            compute(c, bufs[0])

            @pl.when(c + 2 < n_chunks)
            def _():
                fire(c + 2, bufs[0], sem0)

            drain(bufs[1], sem1)
            compute(c + 1, bufs[1])

            @pl.when(c + 3 < n_chunks)
            def _():
                fire(c + 3, bufs[1], sem1)

        pltpu.sync_copy(out_v, out_hbm.at[pl.ds(base, b_per_w)])

    return score_kernel


def kernel(entity_emb, relation_emb, pos_h, pos_r, pos_t):
    B = pos_h.shape[0]
    ent_wide = _tc_transpose(entity_emb.T)
    return _sc_score(B)(ent_wide, relation_emb, pos_h, pos_t, pos_r)
